# baseline (device time: 38996 ns/iter reference)
import jax
import jax.numpy as jnp
from jax import lax
from jax.experimental import pallas as pl
from jax.experimental.pallas import tpu as pltpu

N_DEV = 32


def kernel(A, B):
    m, _ = A.shape
    _, n = B.shape
    rows = m // N_DEV

    def body(a_ref, b_ref, out_ref, partial_ref, rs_buf,
             rs_send_sems, rs_recv_sems, ag_send_sems, ag_recv_sems):
        my = lax.axis_index("i")

        barrier = pltpu.get_barrier_semaphore()
        for k in range(1, N_DEV):
            peer = lax.rem(my + k, N_DEV)
            pl.semaphore_signal(
                barrier, inc=1,
                device_id=(peer,), device_id_type=pl.DeviceIdType.MESH,
            )
        pl.semaphore_wait(barrier, N_DEV - 1)

        partial_ref[...] = jnp.dot(
            a_ref[...], b_ref[...], preferred_element_type=jnp.float32
        )

        rs_sends = []
        for k in range(1, N_DEV):
            dst = lax.rem(my + k, N_DEV)
            rdma = pltpu.make_async_remote_copy(
                src_ref=partial_ref.at[pl.ds(dst * rows, rows), :],
                dst_ref=rs_buf.at[k - 1],
                send_sem=rs_send_sems.at[k - 1],
                recv_sem=rs_recv_sems.at[k - 1],
                device_id=(dst,),
                device_id_type=pl.DeviceIdType.MESH,
            )
            rdma.start()
            rs_sends.append(rdma)

        for j in range(N_DEV - 1):
            pltpu.make_async_remote_copy(
                src_ref=rs_buf.at[j], dst_ref=rs_buf.at[j],
                send_sem=rs_send_sems.at[j], recv_sem=rs_recv_sems.at[j],
                device_id=(my,), device_id_type=pl.DeviceIdType.MESH,
            ).wait_recv()

        z = partial_ref[pl.ds(my * rows, rows), :]
        for j in range(N_DEV - 1):
            z = z + rs_buf[j]
        g = 0.5 * z * (1.0 + jnp.tanh(0.7978845608 * (z + 0.044715 * z * z * z)))
        out_ref[pl.ds(my * rows, rows), :] = g

        ag_sends = []
        for k in range(1, N_DEV):
            dst = lax.rem(my + k, N_DEV)
            rdma = pltpu.make_async_remote_copy(
                src_ref=out_ref.at[pl.ds(my * rows, rows), :],
                dst_ref=out_ref.at[pl.ds(my * rows, rows), :],
                send_sem=ag_send_sems.at[k - 1],
                recv_sem=ag_recv_sems.at[k - 1],
                device_id=(dst,),
                device_id_type=pl.DeviceIdType.MESH,
            )
            rdma.start()
            ag_sends.append(rdma)

        for j in range(N_DEV - 1):
            src_dev = lax.rem(my - (j + 1) + N_DEV, N_DEV)
            pltpu.make_async_remote_copy(
                src_ref=out_ref.at[pl.ds(src_dev * rows, rows), :],
                dst_ref=out_ref.at[pl.ds(src_dev * rows, rows), :],
                send_sem=ag_send_sems.at[j], recv_sem=ag_recv_sems.at[j],
                device_id=(my,), device_id_type=pl.DeviceIdType.MESH,
            ).wait_recv()

        for rdma in rs_sends:
            rdma.wait_send()
        for rdma in ag_sends:
            rdma.wait_send()

    return pl.pallas_call(
        body,
        out_shape=jax.ShapeDtypeStruct((m, n), jnp.float32),
        in_specs=[
            pl.BlockSpec(memory_space=pltpu.VMEM),
            pl.BlockSpec(memory_space=pltpu.VMEM),
        ],
        out_specs=pl.BlockSpec(memory_space=pltpu.VMEM),
        scratch_shapes=[
            pltpu.VMEM((m, n), jnp.float32),
            pltpu.VMEM((N_DEV - 1, m // N_DEV, n), jnp.float32),
            pltpu.SemaphoreType.DMA((N_DEV - 1,)),
            pltpu.SemaphoreType.DMA((N_DEV - 1,)),
            pltpu.SemaphoreType.DMA((N_DEV - 1,)),
            pltpu.SemaphoreType.DMA((N_DEV - 1,)),
        ],
        compiler_params=pltpu.CompilerParams(collective_id=0),
    )(A, B)


# device time: 23951 ns/iter; 1.6282x vs baseline; 1.6282x over previous
import jax
import jax.numpy as jnp
from jax import lax
from jax.experimental import pallas as pl
from jax.experimental.pallas import tpu as pltpu

N_DEV = 32


def kernel(A, B):
    m, _ = A.shape
    _, n = B.shape
    rows = m // N_DEV

    def body(a_ref, b_ref, out_ref, partial_ref, rs_buf,
             rs_send_sems, rs_recv_sems, ag_send_sems, ag_recv_sems):
        my = lax.axis_index("i")

        barrier = pltpu.get_barrier_semaphore()
        for k in range(1, N_DEV):
            peer = lax.rem(my + k, N_DEV)
            pl.semaphore_signal(
                barrier, inc=1,
                device_id=(peer,), device_id_type=pl.DeviceIdType.MESH,
            )

        partial_ref[...] = jnp.dot(
            a_ref[...], b_ref[...], preferred_element_type=jnp.float32
        )

        pl.semaphore_wait(barrier, N_DEV - 1)

        rs_sends = []
        for k in range(1, N_DEV):
            dst = lax.rem(my + k, N_DEV)
            rdma = pltpu.make_async_remote_copy(
                src_ref=partial_ref.at[pl.ds(dst * rows, rows), :],
                dst_ref=rs_buf.at[k - 1],
                send_sem=rs_send_sems.at[k - 1],
                recv_sem=rs_recv_sems.at[k - 1],
                device_id=(dst,),
                device_id_type=pl.DeviceIdType.MESH,
            )
            rdma.start()
            rs_sends.append(rdma)

        z = partial_ref[pl.ds(my * rows, rows), :]
        for j in range(N_DEV - 1):
            pltpu.make_async_remote_copy(
                src_ref=rs_buf.at[j], dst_ref=rs_buf.at[j],
                send_sem=rs_send_sems.at[j], recv_sem=rs_recv_sems.at[j],
                device_id=(my,), device_id_type=pl.DeviceIdType.MESH,
            ).wait_recv()
            z = z + rs_buf[j]
        g = 0.5 * z * (1.0 + jnp.tanh(0.7978845608 * (z + 0.044715 * z * z * z)))
        out_ref[pl.ds(my * rows, rows), :] = g

        ag_sends = []
        for k in range(1, N_DEV):
            dst = lax.rem(my + k, N_DEV)
            rdma = pltpu.make_async_remote_copy(
                src_ref=out_ref.at[pl.ds(my * rows, rows), :],
                dst_ref=out_ref.at[pl.ds(my * rows, rows), :],
                send_sem=ag_send_sems.at[k - 1],
                recv_sem=ag_recv_sems.at[k - 1],
                device_id=(dst,),
                device_id_type=pl.DeviceIdType.MESH,
            )
            rdma.start()
            ag_sends.append(rdma)

        for j in range(N_DEV - 1):
            src_dev = lax.rem(my - (j + 1) + N_DEV, N_DEV)
            pltpu.make_async_remote_copy(
                src_ref=out_ref.at[pl.ds(src_dev * rows, rows), :],
                dst_ref=out_ref.at[pl.ds(src_dev * rows, rows), :],
                send_sem=ag_send_sems.at[j], recv_sem=ag_recv_sems.at[j],
                device_id=(my,), device_id_type=pl.DeviceIdType.MESH,
            ).wait_recv()

        for rdma in rs_sends:
            rdma.wait_send()
        for rdma in ag_sends:
            rdma.wait_send()

    return pl.pallas_call(
        body,
        out_shape=jax.ShapeDtypeStruct((m, n), jnp.float32),
        in_specs=[
            pl.BlockSpec(memory_space=pltpu.VMEM),
            pl.BlockSpec(memory_space=pltpu.VMEM),
        ],
        out_specs=pl.BlockSpec(memory_space=pltpu.VMEM),
        scratch_shapes=[
            pltpu.VMEM((m, n), jnp.float32),
            pltpu.VMEM((N_DEV - 1, m // N_DEV, n), jnp.float32),
            pltpu.SemaphoreType.DMA((N_DEV - 1,)),
            pltpu.SemaphoreType.DMA((N_DEV - 1,)),
            pltpu.SemaphoreType.DMA((N_DEV - 1,)),
            pltpu.SemaphoreType.DMA((N_DEV - 1,)),
        ],
        compiler_params=pltpu.CompilerParams(collective_id=0),
    )(A, B)
